# baseline (speedup 1.0000x reference)
import functools

import jax
import jax.numpy as jnp
from jax import lax
from jax.experimental import pallas as pl
from jax.experimental.pallas import tpu as pltpu
from jax.experimental.pallas import tpu_sc as plsc

_B, _C, _H, _W = 8, 96, 224, 224
_J = 16
_HO, _WO = _H - _J, _W - _J

_OH = [13, 8, 8, 12, 3, 4, 9, 10]
_OW = [5, 1, 6, 11, 6, 15, 0, 11]

_NW = 32
_CPT = _C // _NW
_HH = _HO // 2
_WPAD = _WO + 8
_NL = 16
_KW = _WO // _NL

_mesh = plsc.VectorSubcoreMesh(core_axis_name="c", subcore_axis_name="s")


@functools.partial(
    pl.kernel,
    out_type=jax.ShapeDtypeStruct((_B, _C, _HO, _WO), jnp.float32),
    mesh=_mesh,
    scratch_types=[
        pltpu.VMEM((2, _HH, _WPAD), jnp.float32),
        pltpu.VMEM((2, _HH, _WO), jnp.float32),
        pltpu.SemaphoreType.DMA,
        pltpu.SemaphoreType.DMA,
    ],
    compiler_params=pltpu.CompilerParams(use_tc_tiling_on_sc=False),
)
def _jitter_sc(x_hbm, out_hbm, bin_, bout, gsem, ssem):
    wid = lax.axis_index("s") * 2 + lax.axis_index("c")

    def chunk(t):
        p, q = divmod(t, 2)
        b, i = divmod(p, _CPT)
        oh, ow = _OH[b], _OW[b]
        ow8 = (ow // 8) * 8
        c = wid * _CPT + i
        src = x_hbm.at[b, c, pl.ds(oh + q * _HH, _HH), pl.ds(ow8, _WPAD)]
        dst = out_hbm.at[b, c, pl.ds(q * _HH, _HH)]
        return src, dst, ow - ow8

    def gather(t):
        src, _, _ = chunk(t)
        return pltpu.make_async_copy(src, bin_.at[t % 2], gsem)

    def scatter(t):
        _, dst, _ = chunk(t)
        return pltpu.make_async_copy(bout.at[t % 2], dst, ssem)

    def shift(t):
        _, _, r = chunk(t)
        src2 = bin_.at[t % 2]
        dst2 = bout.at[t % 2]

        @plsc.parallel_loop(0, _HH, unroll=2)
        def row(h):
            vals = [src2[h, pl.ds(r + k * _NL, _NL)] for k in range(_KW)]
            for k in range(_KW):
                dst2[h, pl.ds(k * _NL, _NL)] = vals[k]

    nt = 2 * _CPT * _B
    gather(0).start()
    gather(1).start()
    for t in range(nt):
        gather(t).wait()
        if t >= 2:
            scatter(t - 2).wait()
        shift(t)
        scatter(t).start()
        if t + 2 < nt:
            gather(t + 2).start()
    scatter(nt - 2).wait()
    scatter(nt - 1).wait()


def kernel(x):
    return _jitter_sc(x)


# speedup vs baseline: 3.5182x; 3.5182x over previous
import functools

import jax
import jax.numpy as jnp
from jax import lax
from jax.experimental import pallas as pl
from jax.experimental.pallas import tpu as pltpu
from jax.experimental.pallas import tpu_sc as plsc

_B, _C, _H, _W = 8, 96, 224, 224
_J = 16
_HO, _WO = _H - _J, _W - _J

_OH = [13, 8, 8, 12, 3, 4, 9, 10]
_OW = [5, 1, 6, 11, 6, 15, 0, 11]

_NW = 32
_CPT = _C // _NW
_HH = _HO // 2
_HPAD = _HH + 8
_NL = 16
_KW = _WO // _NL

_mesh = plsc.VectorSubcoreMesh(core_axis_name="c", subcore_axis_name="s")


@functools.partial(
    pl.kernel,
    out_type=jax.ShapeDtypeStruct((_B, _C, _HO, _WO), jnp.float32),
    mesh=_mesh,
    scratch_types=[
        pltpu.VMEM((2, _HPAD, _W), jnp.float32),
        pltpu.VMEM((2, _HH, _WO), jnp.float32),
        pltpu.SemaphoreType.DMA,
        pltpu.SemaphoreType.DMA,
    ],
    compiler_params=pltpu.CompilerParams(use_tc_tiling_on_sc=True),
)
def _jitter_sc(x_hbm, out_hbm, bin_, bout, gsem, ssem):
    wid = lax.axis_index("s") * 2 + lax.axis_index("c")

    def chunk(t):
        p, q = divmod(t, 2)
        b, i = divmod(p, _CPT)
        oh, ow = _OH[b], _OW[b]
        oh8 = (oh // 8) * 8
        c = wid * _CPT + i
        src = x_hbm.at[b, c, pl.ds(oh8 + q * _HH, _HPAD), :]
        dst = out_hbm.at[b, c, pl.ds(q * _HH, _HH)]
        return src, dst, oh - oh8, ow

    def gather(t):
        src, _, _, _ = chunk(t)
        return pltpu.make_async_copy(src, bin_.at[t % 2], gsem)

    def scatter(t):
        _, dst, _, _ = chunk(t)
        return pltpu.make_async_copy(bout.at[t % 2], dst, ssem)

    def shift(t):
        _, _, rh, ow = chunk(t)
        src2 = bin_.at[t % 2]
        dst2 = bout.at[t % 2]

        @plsc.parallel_loop(0, _HH, unroll=2)
        def row(h):
            vals = [src2[rh + h, pl.ds(ow + k * _NL, _NL)] for k in range(_KW)]
            for k in range(_KW):
                dst2[h, pl.ds(k * _NL, _NL)] = vals[k]

    nt = 2 * _CPT * _B
    gather(0).start()
    gather(1).start()
    for t in range(nt):
        gather(t).wait()
        if t >= 2:
            scatter(t - 2).wait()
        shift(t)
        scatter(t).start()
        if t + 2 < nt:
            gather(t + 2).start()
    scatter(nt - 2).wait()
    scatter(nt - 1).wait()


def kernel(x):
    return _jitter_sc(x)
